# baseline (speedup 1.0000x reference)
import functools

import jax
import jax.numpy as jnp
from jax.experimental import pallas as pl
from jax.experimental.pallas import tpu as pltpu
from jax.experimental.pallas import tpu_sc as plsc

DM = 1024
DFF = 2048
NE = 16
TBLK = 128
NT = 32
PADDED = NT * TBLK
FCH = 2
FFB = DFF // FCH


def _router_body(x_ref, wr_ref, idx_ref):
    logits = jax.lax.dot_general(
        x_ref[...], wr_ref[...], (((1,), (1,)), ((), ())),
        preferred_element_type=jnp.float32)
    m = jnp.max(logits, axis=-1, keepdims=True)
    e = jnp.exp(logits - m)
    probs = e / jnp.sum(e, axis=-1, keepdims=True)
    idx_ref[...] = jnp.argmax(probs, axis=-1).astype(jnp.int32)[None, :]


def _router(x_flat, Wr, interpret=False):
    T = x_flat.shape[0]
    return pl.pallas_call(
        _router_body,
        out_shape=jax.ShapeDtypeStruct((1, T), jnp.int32),
        interpret=interpret,
    )(x_flat, Wr)


def _gather_rows(src, ids, interpret=False):
    if interpret:
        return jnp.take(src, ids, axis=0)
    n = ids.shape[0]
    D = src.shape[1]
    info = plsc.get_sparse_core_info()
    nc, ns = info.num_cores, info.num_subcores
    nw = nc * ns
    b_per_w = n // nw
    chunk = min(b_per_w, 64)
    nch = b_per_w // chunk
    mesh = plsc.VectorSubcoreMesh(core_axis_name="c", subcore_axis_name="s")

    @functools.partial(
        pl.kernel, mesh=mesh,
        out_type=jax.ShapeDtypeStruct((n, D), src.dtype),
        scratch_types=[
            pltpu.VMEM((chunk,), jnp.int32),
            pltpu.VMEM((chunk, D), src.dtype),
            pltpu.SemaphoreType.DMA,
        ],
    )
    def k(src_hbm, ids_hbm, out_hbm, idx_v, rows_v, sem):
        wid = jax.lax.axis_index("s") * nc + jax.lax.axis_index("c")
        base = wid * b_per_w
        for c in range(nch):
            off = base + c * chunk
            pltpu.sync_copy(ids_hbm.at[pl.ds(off, chunk)], idx_v)
            pltpu.async_copy(src_hbm.at[idx_v], rows_v, sem).wait()
            pltpu.sync_copy(rows_v, out_hbm.at[pl.ds(off, chunk)])

    return k(src, ids)


def _ffn_body(te_ref, live_ref, xs_ref, w1_ref, b1_ref, w2_ref, b2_ref,
              ys_ref):
    t = pl.program_id(0)

    @pl.when(live_ref[t] > 0)
    def _():
        xb = xs_ref[...].astype(jnp.bfloat16)
        w1b = w1_ref[0].astype(jnp.bfloat16)
        h = jnp.dot(xb, w1b, preferred_element_type=jnp.float32)
        h = h + b1_ref[0]
        h = h * 0.5 * (1.0 + jax.lax.erf(h * 0.7071067811865476))
        w2b = w2_ref[0].astype(jnp.bfloat16)
        y = jnp.dot(h.astype(jnp.bfloat16), w2b,
                    preferred_element_type=jnp.float32)
        ys_ref[...] = y + b2_ref[0]


def _ffn(xs, W1, b1, W2, b2, te, live, interpret=False):
    grid_spec = pltpu.PrefetchScalarGridSpec(
        num_scalar_prefetch=2,
        grid=(NT,),
        in_specs=[
            pl.BlockSpec((TBLK, DM), lambda t, te, lv: (t, 0)),
            pl.BlockSpec((1, DM, DFF), lambda t, te, lv: (te[t], 0, 0)),
            pl.BlockSpec((1, 1, DFF), lambda t, te, lv: (te[t], 0, 0)),
            pl.BlockSpec((1, DFF, DM), lambda t, te, lv: (te[t], 0, 0)),
            pl.BlockSpec((1, 1, DM), lambda t, te, lv: (te[t], 0, 0)),
        ],
        out_specs=pl.BlockSpec((TBLK, DM), lambda t, te, lv: (t, 0)),
    )
    return pl.pallas_call(
        _ffn_body,
        grid_spec=grid_spec,
        out_shape=jax.ShapeDtypeStruct((PADDED, DM), jnp.float32),
        interpret=interpret,
    )(te, live, xs, W1, b1, W2, b2)


def _moe(x, Wr, W1, b1, W2, b2, interpret=False):
    B, T, D = x.shape
    x_flat = x.reshape(B * T, D)
    n_tok = B * T

    idx = _router(x_flat, Wr, interpret=interpret)[0]

    oh = (idx[:, None] == jnp.arange(NE, dtype=jnp.int32)[None, :])
    ohi = oh.astype(jnp.int32)
    cnt = jnp.sum(ohi, axis=0)
    rank = jnp.sum(jnp.cumsum(ohi, axis=0) * ohi, axis=1) - 1
    padded_cnt = ((cnt + TBLK - 1) // TBLK) * TBLK
    gstart = jnp.concatenate(
        [jnp.zeros((1,), jnp.int32),
         jnp.cumsum(padded_cnt)[:-1].astype(jnp.int32)])
    dest = (gstart[idx] + rank).astype(jnp.int32)
    rows = jnp.full((PADDED,), -1, jnp.int32).at[dest].set(
        jnp.arange(n_tok, dtype=jnp.int32))
    first = rows[::TBLK]
    live = (first >= 0).astype(jnp.int32)
    te_raw = idx[jnp.clip(first, 0, n_tok - 1)]
    last_te = jnp.max(jnp.where(live > 0, te_raw, -1)).astype(jnp.int32)
    te = jnp.where(live > 0, te_raw, last_te).astype(jnp.int32)
    rows_c = jnp.maximum(rows, 0)

    xs = _gather_rows(x_flat, rows_c, interpret=interpret)
    ys = _ffn(xs, W1, b1.reshape(NE, 1, DFF), W2, b2.reshape(NE, 1, DM),
              te, live, interpret=interpret)
    out = _gather_rows(ys, dest, interpret=interpret)
    return out.reshape(B, T, D)


def kernel(x, Wr, W1, b1, W2, b2):
    return _moe(x, Wr, W1, b1, W2, b2, interpret=False)


# speedup vs baseline: 1.4899x; 1.4899x over previous
import functools

import jax
import jax.numpy as jnp
from jax.experimental import pallas as pl
from jax.experimental.pallas import tpu as pltpu
from jax.experimental.pallas import tpu_sc as plsc

DM = 1024
DFF = 2048
NE = 16
TBLK = 128
NT = 32
PADDED = NT * TBLK
FCH = 2
FFB = DFF // FCH


def _router_body(x_ref, wr_ref, idx_ref):
    logits = jax.lax.dot_general(
        x_ref[...], wr_ref[...], (((1,), (1,)), ((), ())),
        preferred_element_type=jnp.float32)
    m = jnp.max(logits, axis=-1, keepdims=True)
    e = jnp.exp(logits - m)
    probs = e / jnp.sum(e, axis=-1, keepdims=True)
    idx_ref[...] = jnp.argmax(probs, axis=-1).astype(jnp.int32)[None, :]


def _router(x_flat, Wr, interpret=False):
    T = x_flat.shape[0]
    return pl.pallas_call(
        _router_body,
        out_shape=jax.ShapeDtypeStruct((1, T), jnp.int32),
        interpret=interpret,
    )(x_flat, Wr)


def _gather_rows(src, ids, interpret=False):
    if interpret:
        return jnp.take(src, ids, axis=0)
    n = ids.shape[0]
    D = src.shape[1]
    info = plsc.get_sparse_core_info()
    nc, ns = info.num_cores, info.num_subcores
    nw = nc * ns
    b_per_w = n // nw
    chunk = min(b_per_w, 64)
    nch = b_per_w // chunk
    mesh = plsc.VectorSubcoreMesh(core_axis_name="c", subcore_axis_name="s")

    @functools.partial(
        pl.kernel, mesh=mesh,
        out_type=jax.ShapeDtypeStruct((n, D), src.dtype),
        scratch_types=[
            pltpu.VMEM((chunk,), jnp.int32),
            pltpu.VMEM((chunk, D), src.dtype),
            pltpu.SemaphoreType.DMA,
        ],
    )
    def k(src_hbm, ids_hbm, out_hbm, idx_v, rows_v, sem):
        wid = jax.lax.axis_index("s") * nc + jax.lax.axis_index("c")
        base = wid * b_per_w
        for c in range(nch):
            off = base + c * chunk
            pltpu.sync_copy(ids_hbm.at[pl.ds(off, chunk)], idx_v)
            pltpu.async_copy(src_hbm.at[idx_v], rows_v, sem).wait()
            pltpu.sync_copy(rows_v, out_hbm.at[pl.ds(off, chunk)])

    return k(src, ids)


def _ffn_body(te_ref, live_ref, xs_ref, w1_ref, b1_ref, w2_ref, b2_ref,
              ys_ref):
    t = pl.program_id(0)

    @pl.when(live_ref[t] > 0)
    def _():
        xb = xs_ref[...].astype(jnp.bfloat16)
        w1b = w1_ref[0].astype(jnp.bfloat16)
        h = jnp.dot(xb, w1b, preferred_element_type=jnp.float32)
        h = h + b1_ref[0]
        h = h * 0.5 * (1.0 + jax.lax.erf(h * 0.7071067811865476))
        w2b = w2_ref[0].astype(jnp.bfloat16)
        y = jnp.dot(h.astype(jnp.bfloat16), w2b,
                    preferred_element_type=jnp.float32)
        ys_ref[...] = y + b2_ref[0]


def _ffn(xs, W1, b1, W2, b2, te, live, interpret=False):
    grid_spec = pltpu.PrefetchScalarGridSpec(
        num_scalar_prefetch=2,
        grid=(NT,),
        in_specs=[
            pl.BlockSpec((TBLK, DM), lambda t, te, lv: (t, 0)),
            pl.BlockSpec((1, DM, DFF), lambda t, te, lv: (te[t], 0, 0)),
            pl.BlockSpec((1, 1, DFF), lambda t, te, lv: (te[t], 0, 0)),
            pl.BlockSpec((1, DFF, DM), lambda t, te, lv: (te[t], 0, 0)),
            pl.BlockSpec((1, 1, DM), lambda t, te, lv: (te[t], 0, 0)),
        ],
        out_specs=pl.BlockSpec((TBLK, DM), lambda t, te, lv: (t, 0)),
    )
    return pl.pallas_call(
        _ffn_body,
        grid_spec=grid_spec,
        out_shape=jax.ShapeDtypeStruct((PADDED, DM), jnp.float32),
        interpret=interpret,
    )(te, live, xs, W1, b1, W2, b2)


def _moe(x, Wr, W1, b1, W2, b2, interpret=False):
    B, T, D = x.shape
    x_flat = x.reshape(B * T, D)
    n_tok = B * T

    idx = _router(x_flat, Wr, interpret=interpret)[0]

    oh = (idx[:, None] == jnp.arange(NE, dtype=jnp.int32)[None, :])
    ohi = oh.astype(jnp.int32)
    cnt = jnp.sum(ohi, axis=0)
    rank = jnp.sum(jnp.cumsum(ohi, axis=0) * ohi, axis=1) - 1
    padded_cnt = ((cnt + TBLK - 1) // TBLK) * TBLK
    gstart = jnp.concatenate(
        [jnp.zeros((1,), jnp.int32),
         jnp.cumsum(padded_cnt)[:-1].astype(jnp.int32)])
    dest = (gstart[idx] + rank).astype(jnp.int32)
    rows = jnp.full((PADDED,), -1, jnp.int32).at[dest].set(
        jnp.arange(n_tok, dtype=jnp.int32))
    first = rows[::TBLK]
    live = (first >= 0).astype(jnp.int32)
    te_raw = idx[jnp.clip(first, 0, n_tok - 1)]
    last_te = jnp.max(jnp.where(live > 0, te_raw, -1)).astype(jnp.int32)
    te = jnp.where(live > 0, te_raw, last_te).astype(jnp.int32)
    spread = jnp.arange(PADDED, dtype=jnp.int32) % jnp.int32(n_tok)
    rows_c = jnp.where(rows < 0, spread, rows)

    xs = _gather_rows(x_flat, rows_c, interpret=interpret)
    ys = _ffn(xs, W1, b1.reshape(NE, 1, DFF), W2, b2.reshape(NE, 1, DM),
              te, live, interpret=interpret)
    out = _gather_rows(ys, dest, interpret=interpret)
    return out.reshape(B, T, D)


def kernel(x, Wr, W1, b1, W2, b2):
    return _moe(x, Wr, W1, b1, W2, b2, interpret=False)


# speedup vs baseline: 1.6405x; 1.1011x over previous
import functools

import jax
import jax.numpy as jnp
from jax.experimental import pallas as pl
from jax.experimental.pallas import tpu as pltpu
from jax.experimental.pallas import tpu_sc as plsc

DM = 1024
DFF = 2048
NE = 16
TBLK = 128
NT = 32
PADDED = NT * TBLK
FCH = 2
FFB = DFF // FCH


def _router_body(x_ref, wr_ref, idx_ref):
    logits = jax.lax.dot_general(
        x_ref[...], wr_ref[...], (((1,), (1,)), ((), ())),
        preferred_element_type=jnp.float32)
    m = jnp.max(logits, axis=-1, keepdims=True)
    e = jnp.exp(logits - m)
    probs = e / jnp.sum(e, axis=-1, keepdims=True)
    idx_ref[...] = jnp.argmax(probs, axis=-1).astype(jnp.int32)[None, :]


def _router(x_flat, Wr, interpret=False):
    T = x_flat.shape[0]
    return pl.pallas_call(
        _router_body,
        out_shape=jax.ShapeDtypeStruct((1, T), jnp.int32),
        interpret=interpret,
    )(x_flat, Wr)


def _gather_rows(src, ids, interpret=False):
    if interpret:
        return jnp.take(src, ids, axis=0)
    n = ids.shape[0]
    D = src.shape[1]
    info = plsc.get_sparse_core_info()
    nc, ns = info.num_cores, info.num_subcores
    nw = nc * ns
    b_per_w = n // nw
    chunk = min(b_per_w, 64)
    nch = b_per_w // chunk
    mesh = plsc.VectorSubcoreMesh(core_axis_name="c", subcore_axis_name="s")

    @functools.partial(
        pl.kernel, mesh=mesh,
        out_type=jax.ShapeDtypeStruct((n, D), src.dtype),
        scratch_types=[
            pltpu.VMEM((chunk,), jnp.int32),
            pltpu.VMEM((chunk, D), src.dtype),
            pltpu.SemaphoreType.DMA,
        ],
    )
    def k(src_hbm, ids_hbm, out_hbm, idx_v, rows_v, sem):
        wid = jax.lax.axis_index("s") * nc + jax.lax.axis_index("c")
        base = wid * b_per_w
        for c in range(nch):
            off = base + c * chunk
            pltpu.sync_copy(ids_hbm.at[pl.ds(off, chunk)], idx_v)
            pltpu.async_copy(src_hbm.at[idx_v], rows_v, sem).wait()
            pltpu.sync_copy(rows_v, out_hbm.at[pl.ds(off, chunk)])

    return k(src, ids)


def _sc_scatter_rows(src, dest, out_rows, interpret=False):
    n, D = src.shape
    if interpret:
        return jnp.zeros((out_rows, D), src.dtype).at[dest].set(src)
    info = plsc.get_sparse_core_info()
    nc, ns = info.num_cores, info.num_subcores
    b_per_w = n // (nc * ns)
    mesh = plsc.VectorSubcoreMesh(core_axis_name="c", subcore_axis_name="s")

    @functools.partial(
        pl.kernel, mesh=mesh,
        out_type=jax.ShapeDtypeStruct((out_rows, D), src.dtype),
        scratch_types=[
            pltpu.VMEM((b_per_w,), jnp.int32),
            pltpu.VMEM((b_per_w, D), src.dtype),
            pltpu.SemaphoreType.DMA,
        ],
    )
    def k(src_hbm, dest_hbm, out_hbm, idx_v, rows_v, sem):
        wid = jax.lax.axis_index("s") * nc + jax.lax.axis_index("c")
        base = wid * b_per_w
        pltpu.sync_copy(dest_hbm.at[pl.ds(base, b_per_w)], idx_v)
        pltpu.sync_copy(src_hbm.at[pl.ds(base, b_per_w)], rows_v)
        pltpu.async_copy(rows_v, out_hbm.at[idx_v], sem).wait()

    return k(src, dest)


def _ffn_body(te_ref, live_ref, xs_ref, w1_ref, b1_ref, w2_ref, b2_ref,
              ys_ref):
    t = pl.program_id(0)

    @pl.when(live_ref[t] > 0)
    def _():
        xb = xs_ref[...].astype(jnp.bfloat16)
        w1b = w1_ref[0].astype(jnp.bfloat16)
        h = jnp.dot(xb, w1b, preferred_element_type=jnp.float32)
        h = h + b1_ref[0]
        h = h * 0.5 * (1.0 + jax.lax.erf(h * 0.7071067811865476))
        w2b = w2_ref[0].astype(jnp.bfloat16)
        y = jnp.dot(h.astype(jnp.bfloat16), w2b,
                    preferred_element_type=jnp.float32)
        ys_ref[...] = y + b2_ref[0]


def _ffn(xs, W1, b1, W2, b2, te, live, interpret=False):
    grid_spec = pltpu.PrefetchScalarGridSpec(
        num_scalar_prefetch=2,
        grid=(NT,),
        in_specs=[
            pl.BlockSpec((TBLK, DM), lambda t, te, lv: (t, 0)),
            pl.BlockSpec((1, DM, DFF), lambda t, te, lv: (te[t], 0, 0)),
            pl.BlockSpec((1, 1, DFF), lambda t, te, lv: (te[t], 0, 0)),
            pl.BlockSpec((1, DFF, DM), lambda t, te, lv: (te[t], 0, 0)),
            pl.BlockSpec((1, 1, DM), lambda t, te, lv: (te[t], 0, 0)),
        ],
        out_specs=pl.BlockSpec((TBLK, DM), lambda t, te, lv: (t, 0)),
    )
    return pl.pallas_call(
        _ffn_body,
        grid_spec=grid_spec,
        out_shape=jax.ShapeDtypeStruct((PADDED, DM), jnp.float32),
        interpret=interpret,
    )(te, live, xs, W1, b1, W2, b2)


def _moe(x, Wr, W1, b1, W2, b2, interpret=False):
    B, T, D = x.shape
    x_flat = x.reshape(B * T, D)
    n_tok = B * T

    idx = _router(x_flat, Wr, interpret=interpret)[0]

    oh = (idx[:, None] == jnp.arange(NE, dtype=jnp.int32)[None, :])
    ohi = oh.astype(jnp.int32)
    cnt = jnp.sum(ohi, axis=0)
    rank = jnp.sum(jnp.cumsum(ohi, axis=0) * ohi, axis=1) - 1
    padded_cnt = ((cnt + TBLK - 1) // TBLK) * TBLK
    gstart = jnp.concatenate(
        [jnp.zeros((1,), jnp.int32),
         jnp.cumsum(padded_cnt)[:-1].astype(jnp.int32)])
    dest = (gstart[idx] + rank).astype(jnp.int32)
    ntile = padded_cnt // TBLK
    gtile = gstart // TBLK
    j = jnp.arange(NT, dtype=jnp.int32)[:, None]
    in_e = (j >= gtile[None, :]) & (j < (gtile + ntile)[None, :])
    live = jnp.any(in_e, axis=1).astype(jnp.int32)
    te_raw = jnp.sum(in_e * jnp.arange(NE, dtype=jnp.int32)[None, :], axis=1)
    last_te = jnp.max(jnp.where(live > 0, te_raw, 0)).astype(jnp.int32)
    te = jnp.where(live > 0, te_raw, last_te).astype(jnp.int32)

    xs = _sc_scatter_rows(x_flat, dest, PADDED, interpret=interpret)
    ys = _ffn(xs, W1, b1.reshape(NE, 1, DFF), W2, b2.reshape(NE, 1, DM),
              te, live, interpret=interpret)
    out = _gather_rows(ys, dest, interpret=interpret)
    return out.reshape(B, T, D)


def kernel(x, Wr, W1, b1, W2, b2):
    return _moe(x, Wr, W1, b1, W2, b2, interpret=False)


# speedup vs baseline: 1.8572x; 1.1320x over previous
import functools

import jax
import jax.numpy as jnp
from jax.experimental import pallas as pl
from jax.experimental.pallas import tpu as pltpu
from jax.experimental.pallas import tpu_sc as plsc

DM = 1024
DFF = 2048
NE = 16
TBLK = 128
NT = 32
PADDED = NT * TBLK
FCH = 2
FFB = DFF // FCH


def _router_body(x_ref, wr_ref, dest_ref, telive_ref):
    T = x_ref.shape[0]
    nch = T // TBLK
    logits = jax.lax.dot_general(
        x_ref[...], wr_ref[...], (((1,), (1,)), ((), ())),
        preferred_element_type=jnp.float32)
    m = jnp.max(logits, axis=-1, keepdims=True)
    e = jnp.exp(logits - m)
    probs = e / jnp.sum(e, axis=-1, keepdims=True)
    idx = jnp.argmax(probs, axis=-1).astype(jnp.int32)
    oh = (idx[:, None] == jax.lax.broadcasted_iota(jnp.int32, (T, NE), 1)
          ).astype(jnp.float32)

    r = jax.lax.broadcasted_iota(jnp.int32, (TBLK, TBLK), 0)
    c = jax.lax.broadcasted_iota(jnp.int32, (TBLK, TBLK), 1)
    stril = (r > c).astype(jnp.float32)
    pres = []
    csums = []
    for ci in range(nch):
        oh_c = oh[ci * TBLK:(ci + 1) * TBLK, :]
        pres.append(jax.lax.dot_general(
            stril, oh_c, (((1,), (0,)), ((), ())),
            preferred_element_type=jnp.float32))
        csums.append(jnp.sum(oh_c, axis=0, keepdims=True))
    csum = jnp.concatenate(csums, axis=0)
    rn = jax.lax.broadcasted_iota(jnp.int32, (nch, nch), 0)
    cn = jax.lax.broadcasted_iota(jnp.int32, (nch, nch), 1)
    stril_n = (rn > cn).astype(jnp.float32)
    coff = jax.lax.dot_general(
        stril_n, csum, (((1,), (0,)), ((), ())),
        preferred_element_type=jnp.float32)
    cnt = jnp.sum(csum, axis=0, keepdims=True)

    nt = jnp.floor((cnt + (TBLK - 1.0)) * (1.0 / TBLK))
    re = jax.lax.broadcasted_iota(jnp.int32, (NE, NE), 0)
    ce = jax.lax.broadcasted_iota(jnp.int32, (NE, NE), 1)
    striu_e = (re < ce).astype(jnp.float32)
    gtile = jax.lax.dot_general(
        nt, striu_e, (((1,), (0,)), ((), ())),
        preferred_element_type=jnp.float32)
    gstart = gtile * float(TBLK)

    for ci in range(nch):
        oh_c = oh[ci * TBLK:(ci + 1) * TBLK, :]
        d_c = jnp.sum((pres[ci] + coff[ci:ci + 1, :] + gstart) * oh_c, axis=1)
        dest_ref[ci, :] = d_c.astype(jnp.int32)

    jt = jax.lax.broadcasted_iota(jnp.int32, (NT, NE), 0).astype(jnp.float32)
    in_e = jnp.logical_and(jt >= gtile, jt < gtile + nt).astype(jnp.float32)
    live = jnp.sum(in_e, axis=1)
    te_raw = jnp.sum(
        in_e * jax.lax.broadcasted_iota(jnp.int32, (NT, NE), 1
                                        ).astype(jnp.float32), axis=1)
    last_te = jnp.max(te_raw * live)
    te = jnp.where(live > 0.0, te_raw, last_te)
    telive_ref[0, :] = te.astype(jnp.int32)
    telive_ref[1, :] = live.astype(jnp.int32)


def _router(x_flat, Wr, interpret=False):
    T = x_flat.shape[0]
    return pl.pallas_call(
        _router_body,
        out_shape=(jax.ShapeDtypeStruct((T // TBLK, TBLK), jnp.int32),
                   jax.ShapeDtypeStruct((2, NT), jnp.int32)),
        interpret=interpret,
    )(x_flat, Wr)


def _gather_rows(src, ids, interpret=False):
    if interpret:
        return jnp.take(src, ids, axis=0)
    n = ids.shape[0]
    D = src.shape[1]
    info = plsc.get_sparse_core_info()
    nc, ns = info.num_cores, info.num_subcores
    nw = nc * ns
    b_per_w = n // nw
    chunk = min(b_per_w, 64)
    nch = b_per_w // chunk
    mesh = plsc.VectorSubcoreMesh(core_axis_name="c", subcore_axis_name="s")

    @functools.partial(
        pl.kernel, mesh=mesh,
        out_type=jax.ShapeDtypeStruct((n, D), src.dtype),
        scratch_types=[
            pltpu.VMEM((chunk,), jnp.int32),
            pltpu.VMEM((chunk, D), src.dtype),
            pltpu.SemaphoreType.DMA,
        ],
    )
    def k(src_hbm, ids_hbm, out_hbm, idx_v, rows_v, sem):
        wid = jax.lax.axis_index("s") * nc + jax.lax.axis_index("c")
        base = wid * b_per_w
        for c in range(nch):
            off = base + c * chunk
            pltpu.sync_copy(ids_hbm.at[pl.ds(off, chunk)], idx_v)
            pltpu.async_copy(src_hbm.at[idx_v], rows_v, sem).wait()
            pltpu.sync_copy(rows_v, out_hbm.at[pl.ds(off, chunk)])

    return k(src, ids)


def _sc_scatter_rows(src, dest, out_rows, interpret=False):
    n, D = src.shape
    if interpret:
        return jnp.zeros((out_rows, D), src.dtype).at[dest].set(src)
    info = plsc.get_sparse_core_info()
    nc, ns = info.num_cores, info.num_subcores
    b_per_w = n // (nc * ns)
    mesh = plsc.VectorSubcoreMesh(core_axis_name="c", subcore_axis_name="s")

    @functools.partial(
        pl.kernel, mesh=mesh,
        out_type=jax.ShapeDtypeStruct((out_rows, D), src.dtype),
        scratch_types=[
            pltpu.VMEM((b_per_w,), jnp.int32),
            pltpu.VMEM((b_per_w, D), src.dtype),
            pltpu.SemaphoreType.DMA,
        ],
    )
    def k(src_hbm, dest_hbm, out_hbm, idx_v, rows_v, sem):
        wid = jax.lax.axis_index("s") * nc + jax.lax.axis_index("c")
        base = wid * b_per_w
        pltpu.sync_copy(dest_hbm.at[pl.ds(base, b_per_w)], idx_v)
        pltpu.sync_copy(src_hbm.at[pl.ds(base, b_per_w)], rows_v)
        pltpu.async_copy(rows_v, out_hbm.at[idx_v], sem).wait()

    return k(src, dest)


def _ffn_body(te_ref, live_ref, xs_ref, w1_ref, b1_ref, w2_ref, b2_ref,
              ys_ref):
    t = pl.program_id(0)

    @pl.when(live_ref[t] > 0)
    def _():
        xb = xs_ref[...].astype(jnp.bfloat16)
        w1b = w1_ref[0].astype(jnp.bfloat16)
        h = jnp.dot(xb, w1b, preferred_element_type=jnp.float32)
        h = h + b1_ref[0]
        h = h * 0.5 * (1.0 + jax.lax.erf(h * 0.7071067811865476))
        w2b = w2_ref[0].astype(jnp.bfloat16)
        y = jnp.dot(h.astype(jnp.bfloat16), w2b,
                    preferred_element_type=jnp.float32)
        ys_ref[...] = y + b2_ref[0]


def _ffn(xs, W1, b1, W2, b2, te, live, interpret=False):
    grid_spec = pltpu.PrefetchScalarGridSpec(
        num_scalar_prefetch=2,
        grid=(NT,),
        in_specs=[
            pl.BlockSpec((TBLK, DM), lambda t, te, lv: (t, 0)),
            pl.BlockSpec((1, DM, DFF), lambda t, te, lv: (te[t], 0, 0)),
            pl.BlockSpec((1, 1, DFF), lambda t, te, lv: (te[t], 0, 0)),
            pl.BlockSpec((1, DFF, DM), lambda t, te, lv: (te[t], 0, 0)),
            pl.BlockSpec((1, 1, DM), lambda t, te, lv: (te[t], 0, 0)),
        ],
        out_specs=pl.BlockSpec((TBLK, DM), lambda t, te, lv: (t, 0)),
    )
    return pl.pallas_call(
        _ffn_body,
        grid_spec=grid_spec,
        out_shape=jax.ShapeDtypeStruct((PADDED, DM), jnp.float32),
        interpret=interpret,
    )(te, live, xs, W1, b1, W2, b2)


def _moe(x, Wr, W1, b1, W2, b2, interpret=False):
    B, T, D = x.shape
    x_flat = x.reshape(B * T, D)
    n_tok = B * T

    dest2d, telive = _router(x_flat, Wr, interpret=interpret)
    dest = dest2d.reshape(n_tok)
    te = telive[0]
    live = telive[1]

    xs = _sc_scatter_rows(x_flat, dest, PADDED, interpret=interpret)
    ys = _ffn(xs, W1, b1.reshape(NE, 1, DFF), W2, b2.reshape(NE, 1, DM),
              te, live, interpret=interpret)
    out = _gather_rows(ys, dest, interpret=interpret)
    return out.reshape(B, T, D)


def kernel(x, Wr, W1, b1, W2, b2):
    return _moe(x, Wr, W1, b1, W2, b2, interpret=False)


# speedup vs baseline: 2.0028x; 1.0784x over previous
import functools

import jax
import jax.numpy as jnp
from jax.experimental import pallas as pl
from jax.experimental.pallas import tpu as pltpu
from jax.experimental.pallas import tpu_sc as plsc

DM = 1024
DFF = 2048
NE = 16
TBLK = 256
NT = 24
PADDED = NT * TBLK
FCH = 2
FFB = DFF // FCH


def _router_body(x_ref, wr_ref, dest_ref, telive_ref):
    T = x_ref.shape[0]
    nch = T // TBLK
    logits = jax.lax.dot_general(
        x_ref[...], wr_ref[...], (((1,), (1,)), ((), ())),
        preferred_element_type=jnp.float32)
    m = jnp.max(logits, axis=-1, keepdims=True)
    e = jnp.exp(logits - m)
    probs = e / jnp.sum(e, axis=-1, keepdims=True)
    idx = jnp.argmax(probs, axis=-1).astype(jnp.int32)
    oh = (idx[:, None] == jax.lax.broadcasted_iota(jnp.int32, (T, NE), 1)
          ).astype(jnp.float32)

    r = jax.lax.broadcasted_iota(jnp.int32, (TBLK, TBLK), 0)
    c = jax.lax.broadcasted_iota(jnp.int32, (TBLK, TBLK), 1)
    stril = (r > c).astype(jnp.float32)
    pres = []
    csums = []
    for ci in range(nch):
        oh_c = oh[ci * TBLK:(ci + 1) * TBLK, :]
        pres.append(jax.lax.dot_general(
            stril, oh_c, (((1,), (0,)), ((), ())),
            preferred_element_type=jnp.float32))
        csums.append(jnp.sum(oh_c, axis=0, keepdims=True))
    csum = jnp.concatenate(csums, axis=0)
    rn = jax.lax.broadcasted_iota(jnp.int32, (nch, nch), 0)
    cn = jax.lax.broadcasted_iota(jnp.int32, (nch, nch), 1)
    stril_n = (rn > cn).astype(jnp.float32)
    coff = jax.lax.dot_general(
        stril_n, csum, (((1,), (0,)), ((), ())),
        preferred_element_type=jnp.float32)
    cnt = jnp.sum(csum, axis=0, keepdims=True)

    nt = jnp.floor((cnt + (TBLK - 1.0)) * (1.0 / TBLK))
    re = jax.lax.broadcasted_iota(jnp.int32, (NE, NE), 0)
    ce = jax.lax.broadcasted_iota(jnp.int32, (NE, NE), 1)
    striu_e = (re < ce).astype(jnp.float32)
    gtile = jax.lax.dot_general(
        nt, striu_e, (((1,), (0,)), ((), ())),
        preferred_element_type=jnp.float32)
    gstart = gtile * float(TBLK)

    for ci in range(nch):
        oh_c = oh[ci * TBLK:(ci + 1) * TBLK, :]
        d_c = jnp.sum((pres[ci] + coff[ci:ci + 1, :] + gstart) * oh_c, axis=1)
        dest_ref[ci, :] = d_c.astype(jnp.int32)

    jt = jax.lax.broadcasted_iota(jnp.int32, (NT, NE), 0).astype(jnp.float32)
    in_e = jnp.logical_and(jt >= gtile, jt < gtile + nt).astype(jnp.float32)
    live = jnp.sum(in_e, axis=1)
    te_raw = jnp.sum(
        in_e * jax.lax.broadcasted_iota(jnp.int32, (NT, NE), 1
                                        ).astype(jnp.float32), axis=1)
    last_te = jnp.max(te_raw * live)
    te = jnp.where(live > 0.0, te_raw, last_te)
    telive_ref[0, :] = te.astype(jnp.int32)
    telive_ref[1, :] = live.astype(jnp.int32)


def _router(x_flat, Wr, interpret=False):
    T = x_flat.shape[0]
    return pl.pallas_call(
        _router_body,
        out_shape=(jax.ShapeDtypeStruct((T // TBLK, TBLK), jnp.int32),
                   jax.ShapeDtypeStruct((2, NT), jnp.int32)),
        interpret=interpret,
    )(x_flat, Wr)


def _gather_rows(src, ids, interpret=False):
    if interpret:
        return jnp.take(src, ids, axis=0)
    n = ids.shape[0]
    D = src.shape[1]
    info = plsc.get_sparse_core_info()
    nc, ns = info.num_cores, info.num_subcores
    nw = nc * ns
    b_per_w = n // nw
    chunk = min(b_per_w, 64)
    nch = b_per_w // chunk
    mesh = plsc.VectorSubcoreMesh(core_axis_name="c", subcore_axis_name="s")

    @functools.partial(
        pl.kernel, mesh=mesh,
        out_type=jax.ShapeDtypeStruct((n, D), src.dtype),
        scratch_types=[
            pltpu.VMEM((chunk,), jnp.int32),
            pltpu.VMEM((chunk, D), src.dtype),
            pltpu.SemaphoreType.DMA,
        ],
    )
    def k(src_hbm, ids_hbm, out_hbm, idx_v, rows_v, sem):
        wid = jax.lax.axis_index("s") * nc + jax.lax.axis_index("c")
        base = wid * b_per_w
        for c in range(nch):
            off = base + c * chunk
            pltpu.sync_copy(ids_hbm.at[pl.ds(off, chunk)], idx_v)
            pltpu.async_copy(src_hbm.at[idx_v], rows_v, sem).wait()
            pltpu.sync_copy(rows_v, out_hbm.at[pl.ds(off, chunk)])

    return k(src, ids)


def _sc_scatter_rows(src, dest, out_rows, interpret=False):
    n, D = src.shape
    if interpret:
        return jnp.zeros((out_rows, D), src.dtype).at[dest].set(src)
    info = plsc.get_sparse_core_info()
    nc, ns = info.num_cores, info.num_subcores
    b_per_w = n // (nc * ns)
    mesh = plsc.VectorSubcoreMesh(core_axis_name="c", subcore_axis_name="s")

    @functools.partial(
        pl.kernel, mesh=mesh,
        out_type=jax.ShapeDtypeStruct((out_rows, D), src.dtype),
        scratch_types=[
            pltpu.VMEM((b_per_w,), jnp.int32),
            pltpu.VMEM((b_per_w, D), src.dtype),
            pltpu.SemaphoreType.DMA,
        ],
    )
    def k(src_hbm, dest_hbm, out_hbm, idx_v, rows_v, sem):
        wid = jax.lax.axis_index("s") * nc + jax.lax.axis_index("c")
        base = wid * b_per_w
        pltpu.sync_copy(dest_hbm.at[pl.ds(base, b_per_w)], idx_v)
        pltpu.sync_copy(src_hbm.at[pl.ds(base, b_per_w)], rows_v)
        pltpu.async_copy(rows_v, out_hbm.at[idx_v], sem).wait()

    return k(src, dest)


def _ffn_body(te_ref, live_ref, xs_ref, w1_ref, b1_ref, w2_ref, b2_ref,
              ys_ref):
    t = pl.program_id(0)

    @pl.when(live_ref[t] > 0)
    def _():
        xb = xs_ref[...].astype(jnp.bfloat16)
        w1b = w1_ref[0].astype(jnp.bfloat16)
        h = jnp.dot(xb, w1b, preferred_element_type=jnp.float32)
        h = h + b1_ref[0]
        h = h * 0.5 * (1.0 + jax.lax.erf(h * 0.7071067811865476))
        w2b = w2_ref[0].astype(jnp.bfloat16)
        y = jnp.dot(h.astype(jnp.bfloat16), w2b,
                    preferred_element_type=jnp.float32)
        ys_ref[...] = y + b2_ref[0]


def _ffn(xs, W1, b1, W2, b2, te, live, interpret=False):
    grid_spec = pltpu.PrefetchScalarGridSpec(
        num_scalar_prefetch=2,
        grid=(NT,),
        in_specs=[
            pl.BlockSpec((TBLK, DM), lambda t, te, lv: (t, 0)),
            pl.BlockSpec((1, DM, DFF), lambda t, te, lv: (te[t], 0, 0)),
            pl.BlockSpec((1, 1, DFF), lambda t, te, lv: (te[t], 0, 0)),
            pl.BlockSpec((1, DFF, DM), lambda t, te, lv: (te[t], 0, 0)),
            pl.BlockSpec((1, 1, DM), lambda t, te, lv: (te[t], 0, 0)),
        ],
        out_specs=pl.BlockSpec((TBLK, DM), lambda t, te, lv: (t, 0)),
    )
    return pl.pallas_call(
        _ffn_body,
        grid_spec=grid_spec,
        out_shape=jax.ShapeDtypeStruct((PADDED, DM), jnp.float32),
        interpret=interpret,
    )(te, live, xs, W1, b1, W2, b2)


def _moe(x, Wr, W1, b1, W2, b2, interpret=False):
    B, T, D = x.shape
    x_flat = x.reshape(B * T, D)
    n_tok = B * T

    dest2d, telive = _router(x_flat, Wr, interpret=interpret)
    dest = dest2d.reshape(n_tok)
    te = telive[0]
    live = telive[1]

    xs = _sc_scatter_rows(x_flat, dest, PADDED, interpret=interpret)
    ys = _ffn(xs, W1, b1.reshape(NE, 1, DFF), W2, b2.reshape(NE, 1, DM),
              te, live, interpret=interpret)
    out = _gather_rows(ys, dest, interpret=interpret)
    return out.reshape(B, T, D)


def kernel(x, Wr, W1, b1, W2, b2):
    return _moe(x, Wr, W1, b1, W2, b2, interpret=False)
